# baseline (speedup 1.0000x reference)
import functools

import jax
import jax.numpy as jnp
from jax import lax
from jax.experimental import pallas as pl
from jax.experimental.pallas import tpu as pltpu
from jax.experimental.pallas import tpu_sc as plsc


def _sc_gather_combine(idx1, idx2, entity_embed, relation_embed):
    B = idx1.shape[0]
    D = entity_embed.shape[1]
    H = D // 2
    info = plsc.get_sparse_core_info()
    NC, NS = info.num_cores, info.num_subcores
    NW = NC * NS
    bpw = B // NW

    mesh = plsc.VectorSubcoreMesh(core_axis_name="c", subcore_axis_name="s")

    @functools.partial(
        pl.kernel,
        mesh=mesh,
        compiler_params=pltpu.CompilerParams(use_tc_tiling_on_sc=False),
        out_type=jax.ShapeDtypeStruct((B, D), jnp.float32),
        scratch_types=[
            pltpu.VMEM((bpw,), jnp.int32),
            pltpu.VMEM((bpw,), jnp.int32),
            pltpu.VMEM((bpw, D), jnp.float32),
            pltpu.VMEM((bpw, D), jnp.float32),
            pltpu.VMEM((bpw, D), jnp.float32),
            pltpu.SemaphoreType.DMA,
            pltpu.SemaphoreType.DMA,
        ],
    )
    def body(idx1_hbm, idx2_hbm, ent_hbm, rel_hbm, q_hbm,
             i1_v, i2_v, h_v, r_v, q_v, sem1, sem2):
        wid = lax.axis_index("s") * NC + lax.axis_index("c")
        base = wid * bpw
        pltpu.sync_copy(idx1_hbm.at[pl.ds(base, bpw)], i1_v)
        pltpu.sync_copy(idx2_hbm.at[pl.ds(base, bpw)], i2_v)
        cp_h = pltpu.async_copy(ent_hbm.at[i1_v], h_v, sem1)
        cp_r = pltpu.async_copy(rel_hbm.at[i2_v], r_v, sem2)
        cp_h.wait()
        cp_r.wait()
        for i in range(bpw):
            hr = h_v[i, pl.ds(0, H)]
            hi = h_v[i, pl.ds(H, H)]
            rr = r_v[i, pl.ds(0, H)]
            ri = r_v[i, pl.ds(H, H)]
            q_v[i, pl.ds(0, H)] = hr * rr - hi * ri
            q_v[i, pl.ds(H, H)] = hr * ri + hi * rr
        pltpu.sync_copy(q_v, q_hbm.at[pl.ds(base, bpw)])

    return body(idx1.astype(jnp.int32), idx2.astype(jnp.int32),
                entity_embed, relation_embed)


def _tc_score(q, entity_embed, block_n=4096):
    B, D = q.shape
    N = entity_embed.shape[0]

    def body(e_ref, q_ref, o_ref):
        s = lax.dot_general(e_ref[...], q_ref[...], (((1,), (1,)), ((), ())),
                            preferred_element_type=jnp.float32)
        o_ref[...] = 0.5 * jnp.tanh(0.5 * s) + 0.5

    pT = pl.pallas_call(
        body,
        grid=(pl.cdiv(N, block_n),),
        in_specs=[
            pl.BlockSpec((block_n, D), lambda i: (i, 0)),
            pl.BlockSpec((B, D), lambda i: (0, 0)),
        ],
        out_specs=pl.BlockSpec((block_n, B), lambda i: (i, 0)),
        out_shape=jax.ShapeDtypeStruct((N, B), jnp.float32),
    )(entity_embed, q)
    return pT.T


def kernel(idx1, idx2, entity_embed, relation_embed):
    q = _sc_gather_combine(idx1, idx2, entity_embed, relation_embed)
    return _tc_score(q, entity_embed)


# speedup vs baseline: 1.0028x; 1.0028x over previous
import functools

import jax
import jax.numpy as jnp
from jax import lax
from jax.experimental import pallas as pl
from jax.experimental.pallas import tpu as pltpu
from jax.experimental.pallas import tpu_sc as plsc


def _sc_gather_combine(idx1, idx2, entity_embed, relation_embed):
    B = idx1.shape[0]
    D = entity_embed.shape[1]
    H = D // 2
    info = plsc.get_sparse_core_info()
    NC, NS = info.num_cores, info.num_subcores
    NW = NC * NS
    bpw = B // NW

    mesh = plsc.VectorSubcoreMesh(core_axis_name="c", subcore_axis_name="s")

    @functools.partial(
        pl.kernel,
        mesh=mesh,
        compiler_params=pltpu.CompilerParams(use_tc_tiling_on_sc=False),
        out_type=jax.ShapeDtypeStruct((B, D), jnp.float32),
        scratch_types=[
            pltpu.VMEM((bpw,), jnp.int32),
            pltpu.VMEM((bpw,), jnp.int32),
            pltpu.VMEM((bpw, D), jnp.float32),
            pltpu.VMEM((bpw, D), jnp.float32),
            pltpu.VMEM((bpw, D), jnp.float32),
            pltpu.SemaphoreType.DMA,
            pltpu.SemaphoreType.DMA,
        ],
    )
    def body(idx1_hbm, idx2_hbm, ent_hbm, rel_hbm, q_hbm,
             i1_v, i2_v, h_v, r_v, q_v, sem1, sem2):
        wid = lax.axis_index("s") * NC + lax.axis_index("c")
        base = wid * bpw
        pltpu.sync_copy(idx1_hbm.at[pl.ds(base, bpw)], i1_v)
        pltpu.sync_copy(idx2_hbm.at[pl.ds(base, bpw)], i2_v)
        cp_h = pltpu.async_copy(ent_hbm.at[i1_v], h_v, sem1)
        cp_r = pltpu.async_copy(rel_hbm.at[i2_v], r_v, sem2)
        cp_h.wait()
        cp_r.wait()
        for i in range(bpw):
            hr = h_v[i, pl.ds(0, H)]
            hi = h_v[i, pl.ds(H, H)]
            rr = r_v[i, pl.ds(0, H)]
            ri = r_v[i, pl.ds(H, H)]
            q_v[i, pl.ds(0, H)] = hr * rr - hi * ri
            q_v[i, pl.ds(H, H)] = hr * ri + hi * rr
        pltpu.sync_copy(q_v, q_hbm.at[pl.ds(base, bpw)])

    return body(idx1.astype(jnp.int32), idx2.astype(jnp.int32),
                entity_embed, relation_embed)


def _tc_score(q, entity_embed, block_n=6144):
    B, D = q.shape
    N = entity_embed.shape[0]

    def body(e_ref, q_ref, o_ref):
        s = lax.dot_general(e_ref[...], q_ref[...], (((1,), (1,)), ((), ())),
                            preferred_element_type=jnp.float32)
        o_ref[...] = 0.5 * jnp.tanh(0.5 * s) + 0.5

    pT = pl.pallas_call(
        body,
        grid=(pl.cdiv(N, block_n),),
        in_specs=[
            pl.BlockSpec((block_n, D), lambda i: (i, 0)),
            pl.BlockSpec((B, D), lambda i: (0, 0)),
        ],
        out_specs=pl.BlockSpec((block_n, B), lambda i: (i, 0)),
        out_shape=jax.ShapeDtypeStruct((N, B), jnp.float32),
    )(entity_embed, q)
    return pT.T


def kernel(idx1, idx2, entity_embed, relation_embed):
    q = _sc_gather_combine(idx1, idx2, entity_embed, relation_embed)
    return _tc_score(q, entity_embed)
